# Initial kernel scaffold; baseline (speedup 1.0000x reference)
import jax
import jax.numpy as jnp
from jax.experimental import pallas as pl


def kernel(logit, gt_target, idcs, dim_size, mask):
    raise NotImplementedError("write your pallas kernel here")


# speedup vs baseline: 56.4536x; 56.4536x over previous
import functools

import jax
import jax.numpy as jnp
from jax import lax
from jax.experimental import pallas as pl
from jax.experimental.pallas import tpu as pltpu
from jax.experimental.pallas import tpu_sc as plsc

NC = 2
NS = 16
LANES = 128
CH_ROWS = 32
RB = 2000


_LANE = None


def _shift_up(v, seg_iota, k):
    idx = jnp.maximum(seg_iota - k, 0)
    return lax.gather(
        v, idx[:, None],
        lax.GatherDimensionNumbers(
            offset_dims=(), collapsed_slice_dims=(0,), start_index_map=(0,)),
        slice_sizes=(1,),
        mode=lax.GatherScatterMode.PROMISE_IN_BOUNDS)


def _shift_down(v, seg_iota, k):
    idx = jnp.minimum(seg_iota + k, 15)
    return lax.gather(
        v, idx[:, None],
        lax.GatherDimensionNumbers(
            offset_dims=(), collapsed_slice_dims=(0,), start_index_map=(0,)),
        slice_sizes=(1,),
        mode=lax.GatherScatterMode.PROMISE_IN_BOUNDS)


def _sc_segment_sums(x2, g2, i2, d):
    rows = x2.shape[0]
    n_full = rows // CH_ROWS
    tail = rows - n_full * CH_ROWS
    nw = NC * NS
    outer = (n_full + nw - 1) // nw
    n_rb = d // RB

    mesh = plsc.VectorSubcoreMesh(
        core_axis_name="c", subcore_axis_name="s", num_cores=NC, num_subcores=NS
    )

    @functools.partial(
        pl.kernel,
        out_type=(
            jax.ShapeDtypeStruct((NC * d,), jnp.float32),
            jax.ShapeDtypeStruct((NC * d,), jnp.float32),
            jax.ShapeDtypeStruct((NC * d,), jnp.float32),
            jax.ShapeDtypeStruct((NC * NS * d,), jnp.float32),
        ),
        mesh=mesh,
        compiler_params=pltpu.CompilerParams(needs_layout_passes=False),
        scratch_types=[
            pltpu.VMEM((CH_ROWS, LANES), jnp.float32),
            pltpu.VMEM((CH_ROWS, LANES), jnp.float32),
            pltpu.VMEM((CH_ROWS, LANES), jnp.int32),
            pltpu.VMEM((CH_ROWS, LANES), jnp.float32),
            pltpu.VMEM((CH_ROWS, LANES), jnp.float32),
            pltpu.VMEM((RB,), jnp.float32),
            pltpu.VMEM((d,), jnp.float32),
            pltpu.VMEM_SHARED((d,), jnp.float32),
            pltpu.VMEM_SHARED((d,), jnp.float32),
            pltpu.VMEM_SHARED((d,), jnp.float32),
        ],
    )
    def k(x_hbm, g_hbm, i_hbm, f_out, w_out, g_out, m_out,
          x_b, g_b, i_b, e_b, p_b, tmp, m_loc, f_sh, w_sh, gs_sh):
        cid = lax.axis_index("c")
        sid = lax.axis_index("s")
        wid = sid * NC + cid
        lane = lax.iota(jnp.int32, 16)
        ninf = jnp.full((16,), -jnp.inf, jnp.float32)

        def mrow(i, _):
            m_loc[pl.ds(i * 16, 16)] = ninf
            return 0
        lax.fori_loop(0, d // 16, mrow, 0)

        @pl.when(sid == 0)
        def _init():
            def zrow(i, _):
                tmp[pl.ds(i * 16, 16)] = jnp.zeros((16,), jnp.float32)
                return 0
            lax.fori_loop(0, RB // 16, zrow, 0)

            def zchunk(c, _):
                pltpu.sync_copy(tmp, f_sh.at[pl.ds(c * RB, RB)])
                pltpu.sync_copy(tmp, w_sh.at[pl.ds(c * RB, RB)])
                pltpu.sync_copy(tmp, gs_sh.at[pl.ds(c * RB, RB)])
                return 0
            lax.fori_loop(0, n_rb, zchunk, 0)

        plsc.subcore_barrier()

        def rows_block(n_rows):
            def row(r, __):
                for kk in range(LANES // 16):
                    sl = pl.ds(kk * 16, 16)
                    xv = x_b[r, sl]
                    gv = g_b[r, sl]
                    e_b[r, sl] = jnp.exp(xv)
                    p_b[r, sl] = xv * gv
                    segv = i_b[r, sl]
                    mv = xv
                    for st in (1, 2, 4, 8):
                        mk = _shift_up(mv, lane, st)
                        sk = _shift_up(segv, lane, st)
                        ok = (lane >= st) & (segv == sk)
                        mv = jnp.where(ok, jnp.maximum(mv, mk), mv)
                    seg_next = _shift_down(segv, lane, 1)
                    is_last = (lane == 15) | (segv != seg_next)
                    cur = plsc.load_gather(m_loc, [segv])
                    plsc.store_scatter(m_loc, [segv], jnp.maximum(cur, mv),
                                       mask=is_last)
                pltpu.sync_copy(e_b.at[r], f_sh.at[i_b.at[r]], add=True)
                pltpu.sync_copy(p_b.at[r], w_sh.at[i_b.at[r]], add=True)
                pltpu.sync_copy(g_b.at[r], gs_sh.at[i_b.at[r]], add=True)
                return 0

            lax.fori_loop(0, n_rows, row, 0)

        def do_chunk(t, _):
            c = wid + t * nw

            @pl.when(c < n_full)
            def _():
                base = c * CH_ROWS
                pltpu.sync_copy(x_hbm.at[pl.ds(base, CH_ROWS)], x_b)
                pltpu.sync_copy(g_hbm.at[pl.ds(base, CH_ROWS)], g_b)
                pltpu.sync_copy(i_hbm.at[pl.ds(base, CH_ROWS)], i_b)
                rows_block(CH_ROWS)

            return 0

        lax.fori_loop(0, outer, do_chunk, 0)

        if tail:
            @pl.when(wid == nw - 1)
            def _tail():
                base = n_full * CH_ROWS
                pltpu.sync_copy(x_hbm.at[pl.ds(base, tail)], x_b.at[pl.ds(0, tail)])
                pltpu.sync_copy(g_hbm.at[pl.ds(base, tail)], g_b.at[pl.ds(0, tail)])
                pltpu.sync_copy(i_hbm.at[pl.ds(base, tail)], i_b.at[pl.ds(0, tail)])
                rows_block(tail)

        pltpu.sync_copy(m_loc, m_out.at[pl.ds(wid * d, d)])

        plsc.subcore_barrier()

        for rep in range((n_rb + NS - 1) // NS):
            c = sid + rep * NS

            @pl.when(c < n_rb)
            def _rb():
                off = c * RB
                pltpu.sync_copy(f_sh.at[pl.ds(off, RB)], tmp)
                pltpu.sync_copy(tmp, f_out.at[pl.ds(cid * d + off, RB)])
                pltpu.sync_copy(w_sh.at[pl.ds(off, RB)], tmp)
                pltpu.sync_copy(tmp, w_out.at[pl.ds(cid * d + off, RB)])
                pltpu.sync_copy(gs_sh.at[pl.ds(off, RB)], tmp)
                pltpu.sync_copy(tmp, g_out.at[pl.ds(cid * d + off, RB)])

    return k(x2, g2, i2)


def _tc_finish_body(f0, f1, w0, w1, g0, g1, m_in, sel, o_ref):
    nw = NC * NS
    fsum = f0[...] + f1[...]
    wsum = w0[...] + w1[...]
    gsum = g0[...] + g1[...]
    rows = f0.shape[0]
    m = m_in[...].reshape(nw, rows, LANES).max(axis=0)
    m = jnp.where(jnp.isneginf(m), 0.0, m)
    s = sel[...]
    loss = jnp.where(s > 0.0, jnp.log(fsum) - wsum + m * (gsum - 1.0), 0.0)
    o_ref[0, 0] = jnp.sum(loss) / jnp.sum(s)


def kernel(logit, gt_target, idcs, dim_size, mask):
    n = logit.shape[0]
    d = mask.shape[0]
    rows = n // LANES

    x2 = logit.reshape(rows, LANES)
    g2 = gt_target.reshape(rows, LANES)
    i2 = idcs.reshape(rows, LANES)

    f_flat, w_flat, g_flat, m_flat = _sc_segment_sums(x2, g2, i2, d)
    f_acc = f_flat.reshape(NC, d)
    w_acc = w_flat.reshape(NC, d)
    g_acc = g_flat.reshape(NC, d)
    m_all = m_flat.reshape(NC * NS, d)

    sel = (mask & (jnp.arange(d) < dim_size)).astype(jnp.float32)

    dp = ((d + LANES - 1) // LANES) * LANES
    pad = dp - d
    rows_d = dp // LANES

    def fold(a):
        return jnp.pad(a, (0, pad)).reshape(rows_d, LANES)

    f0, f1 = fold(f_acc[0]), fold(f_acc[1])
    w0, w1 = fold(w_acc[0]), fold(w_acc[1])
    g0, g1 = fold(g_acc[0]), fold(g_acc[1])
    m2 = jnp.pad(m_all, ((0, 0), (0, pad)),
                 constant_values=-jnp.inf).reshape(NC * NS * rows_d, LANES)
    selp = fold(sel)

    out = pl.pallas_call(
        _tc_finish_body,
        out_shape=jax.ShapeDtypeStruct((1, 1), jnp.float32),
        out_specs=pl.BlockSpec(memory_space=pltpu.SMEM),
    )(f0, f1, w0, w1, g0, g1, m2, selp)
    return out[0, 0]
